# baseline (speedup 1.0000x reference)
import functools

import jax
import jax.numpy as jnp
from jax import lax
from jax.experimental import pallas as pl
from jax.experimental.pallas import tpu as pltpu
from jax.experimental.pallas import tpu_sc as plsc

NUM_ENTITIES = 1000000
NUM_RELATIONS = 1000
EMBED_DIM = 32
BATCH = 16384

NC, NS, L = 2, 16, 16
NW = NC * NS
B_PER_W = BATCH // NW
VPB = B_PER_W // L

_mesh = plsc.VectorSubcoreMesh(core_axis_name="c", subcore_axis_name="s")


@functools.partial(
    pl.kernel,
    out_type=jax.ShapeDtypeStruct((BATCH,), jnp.float32),
    mesh=_mesh,
    scratch_types=[
        pltpu.VMEM((B_PER_W,), jnp.int32),
        pltpu.VMEM((B_PER_W,), jnp.int32),
        pltpu.VMEM((B_PER_W,), jnp.int32),
        pltpu.VMEM((EMBED_DIM, NUM_RELATIONS), jnp.float32),
        [pltpu.VMEM((B_PER_W,), jnp.float32) for _ in range(2)],
        [pltpu.VMEM((B_PER_W,), jnp.float32) for _ in range(2)],
        [pltpu.VMEM((B_PER_W,), jnp.int32) for _ in range(2)],
        [pltpu.VMEM((B_PER_W,), jnp.int32) for _ in range(2)],
        pltpu.VMEM((B_PER_W,), jnp.float32),
        [pltpu.SemaphoreType.DMA for _ in range(2)],
        pltpu.SemaphoreType.DMA,
    ],
    compiler_params=pltpu.CompilerParams(needs_layout_passes=False),
)
def _score_kernel(head_hbm, rel_hbm, tail_hbm, ent_hbm, reltab_hbm, out_hbm,
                  hi, ri, ti, relv, hbuf, tbuf, hoff, toff, out_v,
                  sems, rsem):
    wid = lax.axis_index("s") * NC + lax.axis_index("c")
    base = wid * B_PER_W

    pltpu.sync_copy(head_hbm.at[pl.ds(base, B_PER_W)], hi)
    pltpu.sync_copy(rel_hbm.at[pl.ds(base, B_PER_W)], ri)
    pltpu.sync_copy(tail_hbm.at[pl.ds(base, B_PER_W)], ti)
    cp_rel = pltpu.async_copy(reltab_hbm, relv, rsem)

    def issue(j, p):
        off = jnp.int32(j * NUM_ENTITIES)
        for v in range(VPB):
            sl = pl.ds(v * L, L)
            hoff[p][sl] = hi[sl] + off
            toff[p][sl] = ti[sl] + off
        pltpu.async_copy(ent_hbm.at[hoff[p]], hbuf[p], sems[p])
        pltpu.async_copy(ent_hbm.at[toff[p]], tbuf[p], sems[p])

    def wait(j, p):
        pltpu.make_async_copy(ent_hbm.at[hoff[p]], hbuf[p], sems[p]).wait()
        pltpu.make_async_copy(ent_hbm.at[toff[p]], tbuf[p], sems[p]).wait()

    def compute(j, p, acc):
        cols = jnp.full((L,), j, jnp.int32)
        out = []
        for v in range(VPB):
            sl = pl.ds(v * L, L)
            r = plsc.load_gather(relv, [cols, ri[sl]])
            d = (hbuf[p][sl] + r) - tbuf[p][sl]
            out.append(acc[v] + d * d)
        return tuple(out)

    issue(jnp.int32(0), 0)
    issue(jnp.int32(1), 1)
    cp_rel.wait()

    def step(k, acc):
        j0 = 2 * k
        wait(j0, 0)
        acc = compute(j0, 0, acc)

        @pl.when(k < EMBED_DIM // 2 - 1)
        def _():
            issue(j0 + 2, 0)

        j1 = j0 + 1
        wait(j1, 1)
        acc = compute(j1, 1, acc)

        @pl.when(k < EMBED_DIM // 2 - 1)
        def _():
            issue(j0 + 3, 1)

        return acc

    zero = jnp.zeros((L,), jnp.float32)
    acc = lax.fori_loop(0, EMBED_DIM // 2, step,
                        tuple(zero for _ in range(VPB)))

    for v in range(VPB):
        sl = pl.ds(v * L, L)
        x = acc[v] + jnp.float32(1e-12)
        xi = plsc.bitcast(x, jnp.int32)
        yi = jnp.int32(0x5F3759DF) - (xi >> 1)
        y = plsc.bitcast(yi, jnp.float32)
        half_x = jnp.float32(0.5) * x
        for _ in range(3):
            y = y * (jnp.float32(1.5) - half_x * y * y)
        out_v[sl] = -(x * y)

    pltpu.sync_copy(out_v, out_hbm.at[pl.ds(base, B_PER_W)])


def kernel(head_ids, relation_ids, tail_ids, entity_table, relation_table):
    return _score_kernel(
        head_ids.astype(jnp.int32),
        relation_ids.astype(jnp.int32),
        tail_ids.astype(jnp.int32),
        entity_table.T.reshape(-1),
        relation_table.T,
    )


# speedup vs baseline: 1.0025x; 1.0025x over previous
import functools

import jax
import jax.numpy as jnp
from jax import lax
from jax.experimental import pallas as pl
from jax.experimental.pallas import tpu as pltpu
from jax.experimental.pallas import tpu_sc as plsc

NUM_ENTITIES = 1000000
NUM_RELATIONS = 1000
EMBED_DIM = 32
BATCH = 16384

NC, NS, L = 2, 16, 16
NW = NC * NS
B_PER_W = BATCH // NW
VPB = B_PER_W // L

_mesh = plsc.VectorSubcoreMesh(core_axis_name="c", subcore_axis_name="s")


@functools.partial(
    pl.kernel,
    out_type=jax.ShapeDtypeStruct((BATCH,), jnp.float32),
    mesh=_mesh,
    scratch_types=[
        pltpu.VMEM((B_PER_W,), jnp.int32),
        pltpu.VMEM((B_PER_W,), jnp.int32),
        pltpu.VMEM((B_PER_W,), jnp.int32),
        pltpu.VMEM((EMBED_DIM, NUM_RELATIONS), jnp.float32),
        [pltpu.VMEM((B_PER_W,), jnp.float32) for _ in range(2)],
        [pltpu.VMEM((B_PER_W,), jnp.float32) for _ in range(2)],
        pltpu.VMEM((B_PER_W,), jnp.float32),
        [pltpu.SemaphoreType.DMA for _ in range(2)],
        pltpu.SemaphoreType.DMA,
    ],
    compiler_params=pltpu.CompilerParams(
        needs_layout_passes=False, use_tc_tiling_on_sc=False),
)
def _score_kernel(head_hbm, rel_hbm, tail_hbm, ent_hbm, reltab_hbm, out_hbm,
                  hi, ri, ti, relv, hbuf, tbuf, out_v, sems, rsem):
    wid = lax.axis_index("s") * NC + lax.axis_index("c")
    base = wid * B_PER_W

    pltpu.sync_copy(head_hbm.at[pl.ds(base, B_PER_W)], hi)
    pltpu.sync_copy(rel_hbm.at[pl.ds(base, B_PER_W)], ri)
    pltpu.sync_copy(tail_hbm.at[pl.ds(base, B_PER_W)], ti)
    cp_rel = pltpu.async_copy(reltab_hbm, relv, rsem)

    def issue(j, p):
        blk = ent_hbm.at[j]
        pltpu.async_copy(blk.at[hi], hbuf[p], sems[p])
        pltpu.async_copy(blk.at[ti], tbuf[p], sems[p])

    def wait(j, p):
        blk = ent_hbm.at[j]
        pltpu.make_async_copy(blk.at[hi], hbuf[p], sems[p]).wait()
        pltpu.make_async_copy(blk.at[ti], tbuf[p], sems[p]).wait()

    def compute(j, p, acc):
        cols = jnp.full((L,), j, jnp.int32)
        out = []
        for v in range(VPB):
            sl = pl.ds(v * L, L)
            r = plsc.load_gather(relv, [cols, ri[sl]])
            d = (hbuf[p][sl] + r) - tbuf[p][sl]
            out.append(acc[v] + d * d)
        return tuple(out)

    issue(jnp.int32(0), 0)
    issue(jnp.int32(1), 1)
    cp_rel.wait()

    def step(k, acc):
        j0 = 2 * k
        wait(j0, 0)
        acc = compute(j0, 0, acc)

        @pl.when(k < EMBED_DIM // 2 - 1)
        def _():
            issue(j0 + 2, 0)

        j1 = j0 + 1
        wait(j1, 1)
        acc = compute(j1, 1, acc)

        @pl.when(k < EMBED_DIM // 2 - 1)
        def _():
            issue(j0 + 3, 1)

        return acc

    zero = jnp.zeros((L,), jnp.float32)
    acc = lax.fori_loop(0, EMBED_DIM // 2, step,
                        tuple(zero for _ in range(VPB)))

    for v in range(VPB):
        sl = pl.ds(v * L, L)
        x = acc[v] + jnp.float32(1e-12)
        xi = plsc.bitcast(x, jnp.int32)
        yi = jnp.int32(0x5F3759DF) - (xi >> 1)
        y = plsc.bitcast(yi, jnp.float32)
        half_x = jnp.float32(0.5) * x
        for _ in range(3):
            y = y * (jnp.float32(1.5) - half_x * y * y)
        out_v[sl] = -(x * y)

    pltpu.sync_copy(out_v, out_hbm.at[pl.ds(base, B_PER_W)])


def kernel(head_ids, relation_ids, tail_ids, entity_table, relation_table):
    return _score_kernel(
        head_ids.astype(jnp.int32),
        relation_ids.astype(jnp.int32),
        tail_ids.astype(jnp.int32),
        entity_table.T,
        relation_table.T,
    )


# speedup vs baseline: 4.8646x; 4.8526x over previous
import functools

import jax
import jax.numpy as jnp
from jax import lax
from jax.experimental import pallas as pl
from jax.experimental.pallas import tpu as pltpu
from jax.experimental.pallas import tpu_sc as plsc

NUM_ENTITIES = 1000000
NUM_RELATIONS = 1000
EMBED_DIM = 32
BATCH = 16384

NC, NS, L = 2, 16, 16
NW = NC * NS
B_PER_W = BATCH // NW
GROUPS = B_PER_W // L

_mesh = plsc.VectorSubcoreMesh(core_axis_name="c", subcore_axis_name="s")


@functools.partial(
    pl.kernel,
    out_type=jax.ShapeDtypeStruct((BATCH,), jnp.float32),
    mesh=_mesh,
    scratch_types=[
        pltpu.VMEM((B_PER_W,), jnp.int32),
        pltpu.VMEM((B_PER_W,), jnp.int32),
        pltpu.VMEM((B_PER_W,), jnp.int32),
        pltpu.VMEM((B_PER_W, EMBED_DIM), jnp.float32),
        pltpu.VMEM((B_PER_W, EMBED_DIM), jnp.float32),
        pltpu.VMEM((B_PER_W, EMBED_DIM), jnp.float32),
        pltpu.VMEM((B_PER_W,), jnp.float32),
        pltpu.SemaphoreType.DMA,
    ],
    compiler_params=pltpu.CompilerParams(
        needs_layout_passes=False, use_tc_tiling_on_sc=False),
)
def _score_kernel(head_hbm, rel_hbm, tail_hbm, ent_hbm, reltab_hbm, out_hbm,
                  hi, ri, ti, h_rows, r_rows, t_rows, out_v, sem):
    wid = lax.axis_index("s") * NC + lax.axis_index("c")
    base = wid * B_PER_W

    pltpu.sync_copy(head_hbm.at[pl.ds(base, B_PER_W)], hi)
    pltpu.sync_copy(rel_hbm.at[pl.ds(base, B_PER_W)], ri)
    pltpu.sync_copy(tail_hbm.at[pl.ds(base, B_PER_W)], ti)

    cp_h = pltpu.async_copy(ent_hbm.at[hi], h_rows, sem)
    cp_r = pltpu.async_copy(reltab_hbm.at[ri], r_rows, sem)
    cp_t = pltpu.async_copy(ent_hbm.at[ti], t_rows, sem)
    cp_h.wait()
    cp_r.wait()
    cp_t.wait()

    lane = lax.iota(jnp.int32, L)

    def group_body(g, _):
        rows = g * L + lane
        acc = jnp.zeros((L,), jnp.float32)
        for j in range(EMBED_DIM):
            cols = jnp.full((L,), j, jnp.int32)
            h = plsc.load_gather(h_rows, [rows, cols])
            r = plsc.load_gather(r_rows, [rows, cols])
            t = plsc.load_gather(t_rows, [rows, cols])
            d = (h + r) - t
            acc = acc + d * d
        x = acc + jnp.float32(1e-12)
        xi = plsc.bitcast(x, jnp.int32)
        yi = jnp.int32(0x5F3759DF) - (xi >> 1)
        y = plsc.bitcast(yi, jnp.float32)
        half_x = jnp.float32(0.5) * x
        for _ in range(3):
            y = y * (jnp.float32(1.5) - half_x * y * y)
        out_v[pl.ds(g * L, L)] = -(x * y)
        return 0

    lax.fori_loop(0, GROUPS, group_body, 0)
    pltpu.sync_copy(out_v, out_hbm.at[pl.ds(base, B_PER_W)])


def kernel(head_ids, relation_ids, tail_ids, entity_table, relation_table):
    return _score_kernel(
        head_ids.astype(jnp.int32),
        relation_ids.astype(jnp.int32),
        tail_ids.astype(jnp.int32),
        entity_table,
        relation_table,
    )
